# baseline (speedup 1.0000x reference)
import dataclasses
import functools

import jax
import jax.numpy as jnp
from jax import lax
from jax.experimental import pallas as pl
from jax.experimental.pallas import tpu as pltpu
from jax.experimental.pallas import tpu_sc as plsc

E_CNT = 100000
R_CNT = 1000
DIM = 128
B = 16384

NC = 2
NS = 16
L = 16
NW = NC * NS
PER_W = B // NW
C = 16
NCHUNK = PER_W // C


def _sin_poly(x):
    x2 = x * x
    p = x2 * (1.0 / 362880.0) - (1.0 / 5040.0)
    p = x2 * p + (1.0 / 120.0)
    p = x2 * p - (1.0 / 6.0)
    return x + x * (x2 * p)


def _sc_scores(s, o, r, t, e_embed, r_embed, d_frq, h_frq, d_phi, h_phi,
               d_amp, h_amp):
    mesh = plsc.VectorSubcoreMesh(core_axis_name="c", subcore_axis_name="s")

    cp = pltpu.CompilerParams()
    if "needs_layout_passes" in pltpu.CompilerParams.__dataclass_fields__:
        cp = dataclasses.replace(cp, needs_layout_passes=False)

    row_buf = pltpu.VMEM((C, DIM), jnp.float32)
    rel_buf = pltpu.VMEM((C, 2 * DIM), jnp.float32)
    buf_set = [row_buf, row_buf, rel_buf] + [row_buf] * 12
    idx_buf = pltpu.VMEM((C,), jnp.int32)
    t_buf = pltpu.VMEM((C, 2), jnp.float32)

    @functools.partial(
        pl.kernel,
        out_type=jax.ShapeDtypeStruct((B,), jnp.float32),
        mesh=mesh,
        compiler_params=cp,
        scratch_types=[
            *buf_set, *buf_set,
            idx_buf, idx_buf, idx_buf, t_buf,
            idx_buf, idx_buf, idx_buf, t_buf,
            pltpu.VMEM((C,), jnp.float32),
            pltpu.SemaphoreType.DMA,
            pltpu.SemaphoreType.DMA,
        ],
    )
    def k(s_hbm, o_hbm, r_hbm, t_hbm, e_hbm, re_hbm, dfrq_hbm, hfrq_hbm,
          dphi_hbm, hphi_hbm, damp_hbm, hamp_hbm, out_hbm, *rest):
        buf_a = rest[0:15]
        buf_b = rest[15:30]
        quad_a = rest[30:34]
        quad_b = rest[34:38]
        out_c = rest[38]
        sem_g, sem_x = rest[39], rest[40]
        tables = [e_hbm, e_hbm, re_hbm,
                  dfrq_hbm, dphi_hbm, damp_hbm, hfrq_hbm, hphi_hbm, hamp_hbm,
                  dfrq_hbm, dphi_hbm, damp_hbm, hfrq_hbm, hphi_hbm, hamp_hbm]
        sel = [0, 1, 2, 0, 0, 0, 0, 0, 0, 1, 1, 1, 1, 1, 1]

        cid = lax.axis_index("c")
        sid = lax.axis_index("s")
        wid = sid * NC + cid
        base = wid * PER_W

        eidx = lax.iota(jnp.int32, L)

        def load_idx(cb, quad):
            gb = base + cb
            pltpu.async_copy(s_hbm.at[pl.ds(gb, C)], quad[0], sem_x).wait()
            pltpu.async_copy(o_hbm.at[pl.ds(gb, C)], quad[1], sem_x).wait()
            pltpu.async_copy(r_hbm.at[pl.ds(gb, C)], quad[2], sem_x).wait()
            pltpu.async_copy(t_hbm.at[pl.ds(gb, C)], quad[3], sem_x).wait()

        def fire(quad, bufs):
            for tbl, sl, buf in zip(tables, sel, bufs):
                pltpu.async_copy(tbl.at[quad[sl]], buf, sem_g)

        def drain(quad, bufs):
            for tbl, sl, buf in zip(tables, sel, bufs):
                pltpu.make_async_copy(tbl.at[quad[sl]], buf, sem_g).wait()

        def compute(cb, quad, bufs):
            (se_b, oe_b, re_b, sdf_b, sdp_b, sda_b, shf_b, shp_b, sha_b,
             odf_b, odp_b, oda_b, ohf_b, ohp_b, oha_b) = bufs
            t_c = quad[3]
            d_vec = plsc.load_gather(t_c, [eidx, jnp.zeros((L,), jnp.int32)])
            h_vec = plsc.load_gather(t_c, [eidx, jnp.ones((L,), jnp.int32)])

            def body(j, acc):
                jj = jnp.full((L,), j, jnp.int32)
                se = plsc.load_gather(se_b, [eidx, jj])
                oe = plsc.load_gather(oe_b, [eidx, jj])
                r1 = plsc.load_gather(re_b, [eidx, jj])
                r2 = plsc.load_gather(re_b, [eidx, jj + DIM])
                t_s = (plsc.load_gather(sda_b, [eidx, jj])
                       * _sin_poly(plsc.load_gather(sdf_b, [eidx, jj]) * d_vec
                                   + plsc.load_gather(sdp_b, [eidx, jj]))
                       + plsc.load_gather(sha_b, [eidx, jj])
                       * _sin_poly(plsc.load_gather(shf_b, [eidx, jj]) * h_vec
                                   + plsc.load_gather(shp_b, [eidx, jj])))
                t_o = (plsc.load_gather(oda_b, [eidx, jj])
                       * _sin_poly(plsc.load_gather(odf_b, [eidx, jj]) * d_vec
                                   + plsc.load_gather(odp_b, [eidx, jj]))
                       + plsc.load_gather(oha_b, [eidx, jj])
                       * _sin_poly(plsc.load_gather(ohf_b, [eidx, jj]) * h_vec
                                   + plsc.load_gather(ohp_b, [eidx, jj])))
                return (acc + jnp.abs(se + r1 - oe)
                        + jnp.abs(t_s + r2 - t_o))

            acc = plsc.parallel_loop(
                0, DIM, unroll=2, carry=jnp.zeros((L,), jnp.float32))(body)
            out_c[...] = -acc
            pltpu.async_copy(out_c, out_hbm.at[pl.ds(base + cb, C)],
                             sem_x).wait()

        load_idx(0, quad_a)
        fire(quad_a, buf_a)

        @pl.loop(0, NCHUNK - 2, step=2)
        def _chunk(g):
            cb = pl.multiple_of(g * C, C)
            drain(quad_a, buf_a)
            load_idx(cb + C, quad_b)
            fire(quad_b, buf_b)
            compute(cb, quad_a, buf_a)
            drain(quad_b, buf_b)
            load_idx(cb + 2 * C, quad_a)
            fire(quad_a, buf_a)
            compute(cb + C, quad_b, buf_b)

        drain(quad_a, buf_a)
        load_idx((NCHUNK - 1) * C, quad_b)
        fire(quad_b, buf_b)
        compute((NCHUNK - 2) * C, quad_a, buf_a)
        drain(quad_b, buf_b)
        compute((NCHUNK - 1) * C, quad_b, buf_b)

    return k(s, o, r, t, e_embed, r_embed, d_frq, h_frq, d_phi, h_phi,
             d_amp, h_amp)


def kernel(s, o, r, t, e_embed, r_embed, d_frq_embed, h_frq_embed,
           d_phi_embed, h_phi_embed, d_amp_embed, h_amp_embed):
    s = s.astype(jnp.int32)
    o = o.astype(jnp.int32)
    r = r.astype(jnp.int32)
    return _sc_scores(s, o, r, t, e_embed, r_embed, d_frq_embed,
                      h_frq_embed, d_phi_embed, h_phi_embed,
                      d_amp_embed, h_amp_embed)
